# Initial kernel scaffold; baseline (speedup 1.0000x reference)
import jax
import jax.numpy as jnp
from jax.experimental import pallas as pl


def kernel(x, edge_index, W_in, b_in, Wc, bc, gamma, beta, W1, b1, W2, b2):
    raise NotImplementedError("write your pallas kernel here")


# speedup vs baseline: 16.5142x; 16.5142x over previous
import functools

import jax
import jax.numpy as jnp
from jax import lax
from jax.experimental import pallas as pl
from jax.experimental.pallas import tpu as pltpu
from jax.experimental.pallas import tpu_sc as plsc

N = 50000
E = 800000
D_IN = 128
H = 64
BN_EPS = 1e-5

N_PAD = 50176
STRIPE = N_PAD // 16

E_ROWS = 6272
E_PAD = E_ROWS * 128
ROWS_PER_TILE = E_ROWS // 16
CHUNK_ROWS = 4
CHUNKS_PER_TILE = ROWS_PER_TILE // CHUNK_ROWS

DEG_ROWS_PER_TILE = E_ROWS // 32
DEG_BURST = 4
DEG_BURSTS = DEG_ROWS_PER_TILE // DEG_BURST

_MESH = plsc.VectorSubcoreMesh(core_axis_name="c", subcore_axis_name="s")


def _deg_body(dst_hbm, out_hbm, dacc, zbuf, ones, idxb, sem):
    c = lax.axis_index("c")
    s = lax.axis_index("s")
    def zb(i, _):
        zbuf[pl.ds(i * 16, 16)] = jnp.zeros((16,), jnp.float32)
        return _
    lax.fori_loop(0, STRIPE // 16, zb, None)
    for k in range(8):
        ones[pl.ds(k * 16, 16)] = jnp.ones((16,), jnp.float32)
    pltpu.sync_copy(zbuf, dacc.at[pl.ds(s * STRIPE, STRIPE)])
    plsc.subcore_barrier()

    row0 = c * (E_ROWS // 2) + s * DEG_ROWS_PER_TILE

    def burst(i, _):
        r = row0 + i * DEG_BURST
        pltpu.sync_copy(dst_hbm.at[pl.ds(r, DEG_BURST)], idxb)
        cps = [
            pltpu.async_copy(ones, dacc.at[idxb.at[j]], sem, add=True)
            for j in range(DEG_BURST)
        ]
        for cp in cps:
            cp.wait()
        return _

    lax.fori_loop(0, DEG_BURSTS, burst, None)
    plsc.subcore_barrier()
    pltpu.sync_copy(dacc.at[pl.ds(s * STRIPE, STRIPE)], zbuf)
    pltpu.sync_copy(zbuf, out_hbm.at[pl.ds(c * N_PAD + s * STRIPE, STRIPE)])


_deg_kernel = functools.partial(
    pl.kernel,
    out_type=jax.ShapeDtypeStruct((2 * N_PAD,), jnp.float32),
    mesh=_MESH,
    scratch_types=[
        pltpu.VMEM_SHARED((N_PAD + 16,), jnp.float32),
        pltpu.VMEM((STRIPE,), jnp.float32),
        pltpu.VMEM((128,), jnp.float32),
        pltpu.VMEM((DEG_BURST, 128), jnp.int32),
        pltpu.SemaphoreType.DMA,
    ],
)(_deg_body)


def _msg_body(g_hbm, src_hbm, dst_hbm, out_hbm,
              acc, sstage, isrc, idst, rows, gsem, ssem):
    c = lax.axis_index("c")
    s = lax.axis_index("s")
    gbase = c * N_PAD
    _SUB = STRIPE // 8
    for t in range(8):
        pltpu.sync_copy(g_hbm.at[pl.ds(gbase + s * STRIPE + t * _SUB, _SUB)],
                        rows.at[pl.ds(0, _SUB)])
        pltpu.sync_copy(rows.at[pl.ds(0, _SUB)],
                        acc.at[pl.ds(s * STRIPE + t * _SUB, _SUB)])
    plsc.subcore_barrier()

    row0 = s * ROWS_PER_TILE

    def chunk(i, _):
        r = row0 + i * CHUNK_ROWS
        pltpu.sync_copy(src_hbm.at[pl.ds(r, CHUNK_ROWS)], sstage)
        pltpu.sync_copy(dst_hbm.at[pl.ds(r, CHUNK_ROWS)], idst)
        for j in range(CHUNK_ROWS):
            for k in range(8):
                isrc[j, pl.ds(k * 16, 16)] = (
                    sstage[j, pl.ds(k * 16, 16)] + gbase)
        gcps = [
            pltpu.async_copy(g_hbm.at[isrc.at[j]],
                             rows.at[pl.ds(j * 128, 128)], gsem)
            for j in range(CHUNK_ROWS)
        ]
        for cp in gcps:
            cp.wait()
        scps = [
            pltpu.async_copy(rows.at[pl.ds(j * 128, 128)],
                             acc.at[idst.at[j]], ssem, add=True)
            for j in range(CHUNK_ROWS)
        ]
        for cp in scps:
            cp.wait()
        return _

    lax.fori_loop(0, CHUNKS_PER_TILE, chunk, None)
    plsc.subcore_barrier()
    for t in range(8):
        pltpu.sync_copy(acc.at[pl.ds(s * STRIPE + t * _SUB, _SUB)],
                        rows.at[pl.ds(0, _SUB)])
        pltpu.sync_copy(rows.at[pl.ds(0, _SUB)],
                        out_hbm.at[pl.ds(gbase + s * STRIPE + t * _SUB, _SUB)])


_msg_kernel = functools.partial(
    pl.kernel,
    out_type=jax.ShapeDtypeStruct((2 * N_PAD, 32), jnp.float32),
    mesh=_MESH,
    compiler_params=pltpu.CompilerParams(use_tc_tiling_on_sc=False),
    scratch_types=[
        pltpu.VMEM_SHARED((N_PAD, 32), jnp.float32),
        pltpu.VMEM((CHUNK_ROWS, 128), jnp.int32),
        pltpu.VMEM((CHUNK_ROWS, 128), jnp.int32),
        pltpu.VMEM((CHUNK_ROWS, 128), jnp.int32),
        pltpu.VMEM((CHUNK_ROWS * 128, 32), jnp.float32),
        pltpu.SemaphoreType.DMA,
        pltpu.SemaphoreType.DMA,
    ],
)(_msg_body)


_BLK = 400
_GRID = N // _BLK


def _tc_in_body(x_ref, deg_ref, win_ref, bin_ref, wc0_ref,
                g3_ref, h_ref, dinv_ref):
    deg = deg_ref[0] + deg_ref[1] + 1.0
    dinv = lax.rsqrt(deg)
    h = jnp.dot(x_ref[...], win_ref[...],
                preferred_element_type=jnp.float32) + bin_ref[...]
    g = jnp.dot(h, wc0_ref[...], preferred_element_type=jnp.float32) * dinv
    h_ref[...] = h
    dinv_ref[...] = dinv
    g3_ref[0] = g[:, :32]
    g3_ref[1] = g[:, 32:]


def _tc_mid_body(s3_ref, dinv_ref, h_ref, a_ref, b_ref, wc_ref,
                 hnew_ref, g3_ref):
    dinv = dinv_ref[...]
    s_cat = jnp.concatenate([s3_ref[0], s3_ref[1]], axis=1)
    hnew = jnp.maximum(s_cat * dinv * a_ref[...] + b_ref[...], 0.0) \
        + h_ref[...]
    hnew_ref[...] = hnew
    g = jnp.dot(hnew, wc_ref[...], preferred_element_type=jnp.float32) * dinv
    g3_ref[0] = g[:, :32]
    g3_ref[1] = g[:, 32:]


def _tc_out_body(x_ref, s3_ref, dinv_ref, h_ref, a_ref, b_ref,
                 w1_ref, b1_ref, w2_ref, b2_ref, out_ref):
    dinv = dinv_ref[...]
    s_cat = jnp.concatenate([s3_ref[0], s3_ref[1]], axis=1)
    h3 = jnp.maximum(s_cat * dinv * a_ref[...] + b_ref[...], 0.0) + h_ref[...]
    hid = jnp.maximum(
        jnp.dot(h3, w1_ref[...], preferred_element_type=jnp.float32)
        + b1_ref[...], 0.0)
    delta = jnp.dot(hid, w2_ref[...], preferred_element_type=jnp.float32) \
        + b2_ref[...]
    out_ref[...] = x_ref[...] + delta


def _row_spec(cols):
    return pl.BlockSpec((_BLK, cols), lambda i: (i, 0))


def _half_spec():
    return pl.BlockSpec((2, _BLK, 32), lambda i: (0, i, 0))


def _full_spec(r, cols):
    return pl.BlockSpec((r, cols), lambda i: (0, 0))


def kernel(x, edge_index, W_in, b_in, Wc, bc, gamma, beta, W1, b1, W2, b2):
    src = edge_index[0]
    dst = edge_index[1]
    pad = jnp.full((E_PAD - E,), N, dtype=jnp.int32)
    src2d = jnp.concatenate([src, pad]).reshape(E_ROWS, 128)
    dst2d = jnp.concatenate([dst, pad]).reshape(E_ROWS, 128)

    bn_scale = 1.0 / jnp.sqrt(1.0 + BN_EPS)
    A = (gamma * bn_scale).reshape(3, 1, H)
    B = (bc * A.reshape(3, H) + beta).reshape(3, 1, H)

    deg2 = _deg_kernel(dst2d)
    deg3 = deg2.reshape(2, N_PAD, 1)

    g3, h, dinv = pl.pallas_call(
        _tc_in_body,
        grid=(_GRID,),
        in_specs=[
            _row_spec(D_IN),
            pl.BlockSpec((2, _BLK, 1), lambda i: (0, i, 0)),
            _full_spec(D_IN, H),
            _full_spec(1, H),
            _full_spec(H, H),
        ],
        out_specs=[
            _half_spec(),
            _row_spec(H),
            _row_spec(1),
        ],
        out_shape=[
            jax.ShapeDtypeStruct((2, N_PAD, 32), jnp.float32),
            jax.ShapeDtypeStruct((N_PAD, H), jnp.float32),
            jax.ShapeDtypeStruct((N_PAD, 1), jnp.float32),
        ],
    )(x, deg3, W_in, b_in.reshape(1, H), Wc[0])

    for i in range(3):
        s_flat = _msg_kernel(g3.reshape(2 * N_PAD, 32), src2d, dst2d)
        s3 = s_flat.reshape(2, N_PAD, 32)
        if i < 2:
            h, g3 = pl.pallas_call(
                _tc_mid_body,
                grid=(_GRID,),
                in_specs=[
                    _half_spec(),
                    _row_spec(1),
                    _row_spec(H),
                    _full_spec(1, H),
                    _full_spec(1, H),
                    _full_spec(H, H),
                ],
                out_specs=[
                    _row_spec(H),
                    _half_spec(),
                ],
                out_shape=[
                    jax.ShapeDtypeStruct((N_PAD, H), jnp.float32),
                    jax.ShapeDtypeStruct((2, N_PAD, 32), jnp.float32),
                ],
            )(s3, dinv, h, A[i], B[i], Wc[i + 1])
        else:
            out = pl.pallas_call(
                _tc_out_body,
                grid=(_GRID,),
                in_specs=[
                    _row_spec(D_IN),
                    _half_spec(),
                    _row_spec(1),
                    _row_spec(H),
                    _full_spec(1, H),
                    _full_spec(1, H),
                    _full_spec(H, H),
                    _full_spec(1, H),
                    _full_spec(H, D_IN),
                    _full_spec(1, D_IN),
                ],
                out_specs=pl.BlockSpec((_BLK, D_IN), lambda i: (i, 0)),
                out_shape=jax.ShapeDtypeStruct((N, D_IN), jnp.float32),
            )(x, s3, dinv, h, A[2], B[2], W1, b1.reshape(1, H), W2,
              b2.reshape(1, D_IN))
    return out


# speedup vs baseline: 18.7027x; 1.1325x over previous
import functools

import jax
import jax.numpy as jnp
from jax import lax
from jax.experimental import pallas as pl
from jax.experimental.pallas import tpu as pltpu
from jax.experimental.pallas import tpu_sc as plsc

N = 50000
E = 800000
D_IN = 128
H = 64
BN_EPS = 1e-5

N_PAD = 50176
STRIPE = N_PAD // 16

E_ROWS = 6400
E_PAD = E_ROWS * 128
MSG_ROWS = 6336
ROWS_PER_TILE = MSG_ROWS // 16
CHUNK_ROWS = 3
NCH = ROWS_PER_TILE // CHUNK_ROWS

DEG_BURST = 4
DEG_BURSTS = E_ROWS // (32 * DEG_BURST)

_MESH = plsc.VectorSubcoreMesh(core_axis_name="c", subcore_axis_name="s")


def _deg_body(dst_hbm, out_hbm, dacc, zbuf, ones, idxb, sem):
    c = lax.axis_index("c")
    s = lax.axis_index("s")
    def zb(i, _):
        zbuf[pl.ds(i * 16, 16)] = jnp.zeros((16,), jnp.float32)
        return _
    lax.fori_loop(0, STRIPE // 16, zb, None)
    for k in range(8):
        ones[pl.ds(k * 16, 16)] = jnp.ones((16,), jnp.float32)
    pltpu.sync_copy(zbuf, dacc.at[pl.ds(s * STRIPE, STRIPE)])
    plsc.subcore_barrier()

    wid = c * 16 + s

    def burst(i, _):
        r = DEG_BURST * (wid + 32 * i)
        pltpu.sync_copy(dst_hbm.at[pl.ds(r, DEG_BURST)], idxb)
        cps = [
            pltpu.async_copy(ones, dacc.at[idxb.at[j]], sem, add=True)
            for j in range(DEG_BURST)
        ]
        for cp in cps:
            cp.wait()
        return _

    lax.fori_loop(0, DEG_BURSTS, burst, None)
    plsc.subcore_barrier()
    pltpu.sync_copy(dacc.at[pl.ds(s * STRIPE, STRIPE)], zbuf)
    pltpu.sync_copy(zbuf, out_hbm.at[pl.ds(c * N_PAD + s * STRIPE, STRIPE)])


_deg_kernel = functools.partial(
    pl.kernel,
    out_type=jax.ShapeDtypeStruct((2 * N_PAD,), jnp.float32),
    mesh=_MESH,
    scratch_types=[
        pltpu.VMEM_SHARED((N_PAD + 16,), jnp.float32),
        pltpu.VMEM((STRIPE,), jnp.float32),
        pltpu.VMEM((128,), jnp.float32),
        pltpu.VMEM((DEG_BURST, 128), jnp.int32),
        pltpu.SemaphoreType.DMA,
    ],
)(_deg_body)


def _msg_body(g_hbm, src_hbm, dst_hbm, out_hbm,
              acc, ss0, ss1, id0, id1, rw0, rw1, gs0, gs1, sc0, sc1):
    c = lax.axis_index("c")
    s = lax.axis_index("s")
    gbase = c * N_PAD
    sstage = (ss0, ss1)
    idst = (id0, id1)
    rows = (rw0, rw1)
    gsem = (gs0, gs1)
    ssem = (sc0, sc1)
    _SUB = 224
    for t in range(STRIPE // (2 * _SUB)):
        for b in range(2):
            u = 2 * t + b
            pltpu.sync_copy(
                g_hbm.at[pl.ds(gbase + s * STRIPE + u * _SUB, _SUB)],
                rows[b].at[pl.ds(0, _SUB)])
            pltpu.sync_copy(rows[b].at[pl.ds(0, _SUB)],
                            acc.at[pl.ds(s * STRIPE + u * _SUB, _SUB)])
    plsc.subcore_barrier()

    row0 = s * ROWS_PER_TILE

    def stage_and_gather(b, i):
        r = row0 + i * CHUNK_ROWS
        pltpu.sync_copy(src_hbm.at[pl.ds(r, CHUNK_ROWS)], sstage[b])
        pltpu.sync_copy(dst_hbm.at[pl.ds(r, CHUNK_ROWS)], idst[b])
        for j in range(CHUNK_ROWS):
            for k in range(8):
                sl = pl.ds(k * 16, 16)
                sstage[b][j, sl] = sstage[b][j, sl] + gbase
        for j in range(CHUNK_ROWS):
            pltpu.async_copy(g_hbm.at[sstage[b].at[j]],
                             rows[b].at[pl.ds(j * 128, 128)], gsem[b])

    def wait_gather_issue_scatter(b):
        for j in range(CHUNK_ROWS):
            pltpu.make_async_copy(g_hbm.at[sstage[b].at[j]],
                                  rows[b].at[pl.ds(j * 128, 128)],
                                  gsem[b]).wait()
        for j in range(CHUNK_ROWS):
            pltpu.async_copy(rows[b].at[pl.ds(j * 128, 128)],
                             acc.at[idst[b].at[j]], ssem[b], add=True)

    def wait_scatter(b):
        for j in range(CHUNK_ROWS):
            pltpu.make_async_copy(rows[b].at[pl.ds(j * 128, 128)],
                                  acc.at[idst[b].at[j]], ssem[b]).wait()

    def iter_body(i, carry):
        for p in range(2):
            @pl.when(i % 2 == p)
            def _parity():
                @pl.when(i >= 2)
                def _w():
                    wait_scatter(p)
                @pl.when(i < NCH)
                def _g():
                    stage_and_gather(p, i)
                @pl.when(i >= 1)
                def _s():
                    wait_gather_issue_scatter(1 - p)
        return carry

    lax.fori_loop(0, NCH + 1, iter_body, None)
    wait_scatter((NCH - 1) % 2)
    plsc.subcore_barrier()
    for t in range(STRIPE // (2 * _SUB)):
        for b in range(2):
            u = 2 * t + b
            pltpu.sync_copy(acc.at[pl.ds(s * STRIPE + u * _SUB, _SUB)],
                            rows[b].at[pl.ds(0, _SUB)])
            pltpu.sync_copy(
                rows[b].at[pl.ds(0, _SUB)],
                out_hbm.at[pl.ds(gbase + s * STRIPE + u * _SUB, _SUB)])


_msg_kernel = functools.partial(
    pl.kernel,
    out_type=jax.ShapeDtypeStruct((2 * N_PAD, 32), jnp.float32),
    mesh=_MESH,
    compiler_params=pltpu.CompilerParams(use_tc_tiling_on_sc=False),
    scratch_types=[
        pltpu.VMEM_SHARED((N_PAD, 32), jnp.float32),
        pltpu.VMEM((CHUNK_ROWS, 128), jnp.int32),
        pltpu.VMEM((CHUNK_ROWS, 128), jnp.int32),
        pltpu.VMEM((CHUNK_ROWS, 128), jnp.int32),
        pltpu.VMEM((CHUNK_ROWS, 128), jnp.int32),
        pltpu.VMEM((CHUNK_ROWS * 128, 32), jnp.float32),
        pltpu.VMEM((CHUNK_ROWS * 128, 32), jnp.float32),
        pltpu.SemaphoreType.DMA,
        pltpu.SemaphoreType.DMA,
        pltpu.SemaphoreType.DMA,
        pltpu.SemaphoreType.DMA,
    ],
)(_msg_body)


_BLK = 2000
_GRID = N // _BLK


def _tc_in_body(x_ref, deg_ref, win_ref, bin_ref, wc0_ref,
                g3_ref, h_ref, dinv_ref):
    deg = deg_ref[0] + deg_ref[1] + 1.0
    dinv = lax.rsqrt(deg)
    h = jnp.dot(x_ref[...], win_ref[...],
                preferred_element_type=jnp.float32) + bin_ref[...]
    g = jnp.dot(h, wc0_ref[...], preferred_element_type=jnp.float32) * dinv
    h_ref[...] = h
    dinv_ref[...] = dinv
    g3_ref[0] = g[:, :32]
    g3_ref[1] = g[:, 32:]


def _tc_mid_body(s3_ref, dinv_ref, h_ref, a_ref, b_ref, wc_ref,
                 hnew_ref, g3_ref):
    dinv = dinv_ref[...]
    s_cat = jnp.concatenate([s3_ref[0], s3_ref[1]], axis=1)
    hnew = jnp.maximum(s_cat * dinv * a_ref[...] + b_ref[...], 0.0) \
        + h_ref[...]
    hnew_ref[...] = hnew
    g = jnp.dot(hnew, wc_ref[...], preferred_element_type=jnp.float32) * dinv
    g3_ref[0] = g[:, :32]
    g3_ref[1] = g[:, 32:]


def _tc_out_body(x_ref, s3_ref, dinv_ref, h_ref, a_ref, b_ref,
                 w1_ref, b1_ref, w2_ref, b2_ref, out_ref):
    dinv = dinv_ref[...]
    s_cat = jnp.concatenate([s3_ref[0], s3_ref[1]], axis=1)
    h3 = jnp.maximum(s_cat * dinv * a_ref[...] + b_ref[...], 0.0) + h_ref[...]
    hid = jnp.maximum(
        jnp.dot(h3, w1_ref[...], preferred_element_type=jnp.float32)
        + b1_ref[...], 0.0)
    delta = jnp.dot(hid, w2_ref[...], preferred_element_type=jnp.float32) \
        + b2_ref[...]
    out_ref[...] = x_ref[...] + delta


def _row_spec(cols):
    return pl.BlockSpec((_BLK, cols), lambda i: (i, 0))


def _half_spec():
    return pl.BlockSpec((2, _BLK, 32), lambda i: (0, i, 0))


def _full_spec(r, cols):
    return pl.BlockSpec((r, cols), lambda i: (0, 0))


def kernel(x, edge_index, W_in, b_in, Wc, bc, gamma, beta, W1, b1, W2, b2):
    src = edge_index[0]
    dst = edge_index[1]
    pad = jnp.full((E_PAD - E,), N, dtype=jnp.int32)
    src2d = jnp.concatenate([src, pad]).reshape(E_ROWS, 128)
    dst2d = jnp.concatenate([dst, pad]).reshape(E_ROWS, 128)

    bn_scale = 1.0 / jnp.sqrt(1.0 + BN_EPS)
    A = (gamma * bn_scale).reshape(3, 1, H)
    B = (bc * A.reshape(3, H) + beta).reshape(3, 1, H)

    deg2 = _deg_kernel(dst2d)
    deg3 = deg2.reshape(2, N_PAD, 1)

    g3, h, dinv = pl.pallas_call(
        _tc_in_body,
        grid=(_GRID,),
        in_specs=[
            _row_spec(D_IN),
            pl.BlockSpec((2, _BLK, 1), lambda i: (0, i, 0)),
            _full_spec(D_IN, H),
            _full_spec(1, H),
            _full_spec(H, H),
        ],
        out_specs=[
            _half_spec(),
            _row_spec(H),
            _row_spec(1),
        ],
        out_shape=[
            jax.ShapeDtypeStruct((2, N_PAD, 32), jnp.float32),
            jax.ShapeDtypeStruct((N_PAD, H), jnp.float32),
            jax.ShapeDtypeStruct((N_PAD, 1), jnp.float32),
        ],
    )(x, deg3, W_in, b_in.reshape(1, H), Wc[0])

    for i in range(3):
        s_flat = _msg_kernel(g3.reshape(2 * N_PAD, 32), src2d, dst2d)
        s3 = s_flat.reshape(2, N_PAD, 32)
        if i < 2:
            h, g3 = pl.pallas_call(
                _tc_mid_body,
                grid=(_GRID,),
                in_specs=[
                    _half_spec(),
                    _row_spec(1),
                    _row_spec(H),
                    _full_spec(1, H),
                    _full_spec(1, H),
                    _full_spec(H, H),
                ],
                out_specs=[
                    _row_spec(H),
                    _half_spec(),
                ],
                out_shape=[
                    jax.ShapeDtypeStruct((N_PAD, H), jnp.float32),
                    jax.ShapeDtypeStruct((2, N_PAD, 32), jnp.float32),
                ],
            )(s3, dinv, h, A[i], B[i], Wc[i + 1])
        else:
            out = pl.pallas_call(
                _tc_out_body,
                grid=(_GRID,),
                in_specs=[
                    _row_spec(D_IN),
                    _half_spec(),
                    _row_spec(1),
                    _row_spec(H),
                    _full_spec(1, H),
                    _full_spec(1, H),
                    _full_spec(H, H),
                    _full_spec(1, H),
                    _full_spec(H, D_IN),
                    _full_spec(1, D_IN),
                ],
                out_specs=pl.BlockSpec((_BLK, D_IN), lambda i: (i, 0)),
                out_shape=jax.ShapeDtypeStruct((N, D_IN), jnp.float32),
            )(x, s3, dinv, h, A[2], B[2], W1, b1.reshape(1, H), W2,
              b2.reshape(1, D_IN))
    return out
